# baseline (speedup 1.0000x reference)
import functools

import jax
import jax.numpy as jnp
from jax import lax
from jax.experimental import pallas as pl
from jax.experimental.pallas import tpu as pltpu
from jax.experimental.pallas import tpu_sc as plsc

NC = 2
NS = 16
NW = NC * NS
L = 16

B = 16384
D = 64
CTX = 20
CH = 32
BPW = B // NW
NCHUNK = BPW // CH


def _mean_body(idx_ref, vt_hbm, o_ref, blks, sem):
    cps = []
    for j in range(2 * CTX):
        c0 = (idx_ref[j] // 128) * 128
        cps.append(pltpu.async_copy(vt_hbm.at[:, pl.ds(c0, 128)],
                                    blks.at[j], sem))
    for cp in cps:
        cp.wait()
    data = blks[...]
    lane = lax.broadcasted_iota(jnp.int32, (2 * CTX, 1, 128), 2)
    cols = jnp.zeros((2 * CTX, 1, 128), jnp.int32)
    for j in range(2 * CTX):
        cols = cols + jnp.where(
            lax.broadcasted_iota(jnp.int32, (2 * CTX, 1, 128), 0) == j,
            idx_ref[j] % 128, 0)
    picked = jnp.sum(jnp.where(lane == cols, data, 0.0), axis=2)
    o_ref[0, :] = jnp.sum(picked[:CTX], axis=0) * (1.0 / CTX)
    o_ref[1, :] = jnp.sum(picked[CTX:], axis=0) * (1.0 / CTX)


_mean = pl.pallas_call(
    _mean_body,
    grid_spec=pltpu.PrefetchScalarGridSpec(
        num_scalar_prefetch=1,
        in_specs=[pl.BlockSpec(memory_space=pl.ANY)],
        out_specs=pl.BlockSpec((2, D), lambda idx_ref: (0, 0)),
        scratch_shapes=[pltpu.VMEM((2 * CTX, D, 128), jnp.float32),
                        pltpu.SemaphoreType.DMA],
    ),
    out_shape=jax.ShapeDtypeStruct((2, D), jnp.float32),
)


def _fire_chunk(utab, idx_ref, ioff, buf, sem):
    def body(g, carry):
        vec = idx_ref[pl.ds(ioff + g * L, L)]
        bvec = vec & (-16)
        for k in range(L):
            src = pl.multiple_of(bvec[k], 16)
            dst = pl.multiple_of((g * L + k) * 16, 16)
            pltpu.async_copy(utab.at[pl.ds(src, 16), :],
                             buf.at[pl.ds(dst, 16), :], sem)
        return carry
    lax.fori_loop(0, CH // L, body, 0)


def _dot_chunk(buf, idx_ref, ioff, m, masks, zref, zoff):
    def group(g, carry):
        vec = idx_ref[pl.ds(ioff + g * L, L)]
        svec = vec & 15
        z = jnp.zeros((L,), jnp.float32)
        for j in range(L):
            row = (g * L + j) * 16 + svec[j]
            pa = jnp.zeros((L,), jnp.float32)
            for t in range(2):
                a, b = plsc.unpack(buf[row, pl.ds(t * 2 * L, 2 * L)],
                                   format=plsc.PackFormat.INTERLEAVED)
                pa = pa + a * m[2 * t] + b * m[2 * t + 1]
            z = jnp.where(masks[j], jnp.sum(pa), z)
        zref[pl.ds(zoff + g * L, L)] = z
        return carry
    lax.fori_loop(0, CH // L, group, 0)


def _stage_a_body(means, posu, negu, utab,
                  zpos_out, zneg_out,
                  uidx_v, mv, bufa, bufb, zp, zn, sema, semb):
    bufs = [bufa, bufb]
    sems = [sema, semb]
    wid = lax.axis_index("s") * NC + lax.axis_index("c")
    base = wid * BPW

    pltpu.sync_copy(means, mv)
    pltpu.sync_copy(posu.at[pl.ds(base, BPW)], uidx_v.at[pl.ds(0, BPW)])
    pltpu.sync_copy(negu.at[pl.ds(base, BPW)], uidx_v.at[pl.ds(BPW, BPW)])

    m_pos = [mv[0, pl.ds(k * L, L)] for k in range(D // L)]
    m_neg = [mv[1, pl.ds(k * L, L)] for k in range(D // L)]
    iota16 = lax.iota(jnp.int32, L)
    masks = [iota16 == j for j in range(L)]

    def run_table(toff, m, zref):
        _fire_chunk(utab, uidx_v, toff, bufs[0], sems[0])
        _fire_chunk(utab, uidx_v, toff + CH, bufs[1], sems[1])

        def step(i, carry):
            for half in (0, 1):
                c = 2 * i + half
                pltpu.make_async_copy(utab.at[pl.ds(0, CH * 16), :],
                                      bufs[half], sems[half]).wait()
                _dot_chunk(bufs[half], uidx_v, toff + c * CH, m, masks,
                           zref, c * CH)

                @pl.when(c + 2 < NCHUNK)
                def _():
                    _fire_chunk(utab, uidx_v, toff + (c + 2) * CH,
                                bufs[half], sems[half])
            return carry
        lax.fori_loop(0, NCHUNK // 2, step, 0)

    run_table(0, m_pos, zp)
    pltpu.sync_copy(zp, zpos_out.at[pl.ds(base, BPW)])
    run_table(BPW, m_neg, zn)
    pltpu.sync_copy(zn, zneg_out.at[pl.ds(base, BPW)])


_stage_a = functools.partial(
    pl.kernel,
    out_type=(jax.ShapeDtypeStruct((B,), jnp.float32),
              jax.ShapeDtypeStruct((B,), jnp.float32)),
    mesh=plsc.VectorSubcoreMesh(core_axis_name="c", subcore_axis_name="s",
                                num_cores=NC, num_subcores=NS),
    compiler_params=pltpu.CompilerParams(needs_layout_passes=False,
                                         use_tc_tiling_on_sc=True),
    scratch_types=[
        pltpu.VMEM((2 * BPW,), jnp.int32),
        pltpu.VMEM((2, D), jnp.float32),
        pltpu.VMEM((CH * 16, D), jnp.bfloat16),
        pltpu.VMEM((CH * 16, D), jnp.bfloat16),
        pltpu.VMEM((BPW,), jnp.float32),
        pltpu.VMEM((BPW,), jnp.float32),
        pltpu.SemaphoreType.DMA,
        pltpu.SemaphoreType.DMA,
    ],
)(_stage_a_body)


def _loss_body(pz_ref, nz_ref, o_ref):
    def logsig(x):
        return jnp.minimum(x, 0.0) - jnp.log1p(jnp.exp(-jnp.abs(x)))
    total = -(jnp.sum(logsig(pz_ref[...])) + jnp.sum(logsig(-nz_ref[...])))
    o_ref[...] = jnp.reshape(total, (1, 1))


_loss = pl.pallas_call(
    _loss_body,
    out_shape=jax.ShapeDtypeStruct((1, 1), jnp.float32),
)


def kernel(pos_v, pos_u, neg_v, neg_u, v_table, u_table):
    vidx = jnp.concatenate([pos_v[-1], neg_v[-1]])
    means = _mean(vidx, v_table.T)
    mperm = jnp.concatenate(
        [means[:, 0:32:2], means[:, 1:32:2], means[:, 32:64:2],
         means[:, 33:64:2]], axis=1)
    zp, zn = _stage_a(mperm, pos_u, neg_u, u_table.astype(jnp.bfloat16))
    out = _loss(zp.reshape(B // 128, 128), zn.reshape(B // 128, 128))
    return out[0, 0]


# speedup vs baseline: 1.0449x; 1.0449x over previous
import functools

import jax
import jax.numpy as jnp
from jax import lax
from jax.experimental import pallas as pl
from jax.experimental.pallas import tpu as pltpu
from jax.experimental.pallas import tpu_sc as plsc

NC = 2
NS = 16
NW = NC * NS
L = 16

B = 16384
D = 64
CTX = 20
CH = 32
BPW = B // NW
NCHUNK = BPW // CH


def _mean_body(idx_ref, vt_hbm, o_ref, blks, sem):
    cps = []
    for j in range(2 * CTX):
        c0 = (idx_ref[j] // 128) * 128
        cps.append(pltpu.async_copy(vt_hbm.at[:, pl.ds(c0, 128)],
                                    blks.at[j], sem))
    for cp in cps:
        cp.wait()
    data = blks[...]
    lane = lax.broadcasted_iota(jnp.int32, (2 * CTX, 1, 128), 2)
    cols = jnp.zeros((2 * CTX, 1, 128), jnp.int32)
    for j in range(2 * CTX):
        cols = cols + jnp.where(
            lax.broadcasted_iota(jnp.int32, (2 * CTX, 1, 128), 0) == j,
            idx_ref[j] % 128, 0)
    picked = jnp.sum(jnp.where(lane == cols, data, 0.0), axis=2)
    o_ref[0, :] = jnp.sum(picked[:CTX], axis=0) * (1.0 / CTX)
    o_ref[1, :] = jnp.sum(picked[CTX:], axis=0) * (1.0 / CTX)


_mean = pl.pallas_call(
    _mean_body,
    grid_spec=pltpu.PrefetchScalarGridSpec(
        num_scalar_prefetch=1,
        in_specs=[pl.BlockSpec(memory_space=pl.ANY)],
        out_specs=pl.BlockSpec((2, D), lambda idx_ref: (0, 0)),
        scratch_shapes=[pltpu.VMEM((2 * CTX, D, 128), jnp.float32),
                        pltpu.SemaphoreType.DMA],
    ),
    out_shape=jax.ShapeDtypeStruct((2, D), jnp.float32),
)


def _fire_chunk(utab, idx_ref, ioff, buf, sem):
    def body(g, carry):
        vec = idx_ref[pl.ds(ioff + g * L, L)]
        bvec = vec & (-8)
        for k in range(L):
            src = pl.multiple_of(bvec[k], 8)
            dst = pl.multiple_of((g * L + k) * 8, 8)
            pltpu.async_copy(utab.at[pl.ds(src, 8), :],
                             buf.at[pl.ds(dst, 8), :], sem)
        return carry
    lax.fori_loop(0, CH // L, body, 0)


def _dot_chunk(buf, idx_ref, ioff, m, masks, zref, zoff):
    def group(g, carry):
        vec = idx_ref[pl.ds(ioff + g * L, L)]
        svec = vec & 7
        z = jnp.zeros((L,), jnp.float32)
        for j in range(L):
            row = (g * L + j) * 8 + svec[j]
            pa = buf[row, pl.ds(0, L)] * m[0]
            for k in range(1, D // L):
                pa = pa + buf[row, pl.ds(k * L, L)] * m[k]
            z = jnp.where(masks[j], jnp.sum(pa), z)
        zref[pl.ds(zoff + g * L, L)] = z
        return carry
    lax.fori_loop(0, CH // L, group, 0)


def _stage_a_body(means, posu, negu, utab,
                  zpos_out, zneg_out,
                  uidx_v, mv, bufa, bufb, zp, zn, sema, semb):
    bufs = [bufa, bufb]
    sems = [sema, semb]
    wid = lax.axis_index("s") * NC + lax.axis_index("c")
    base = wid * BPW

    pltpu.sync_copy(means, mv)
    pltpu.sync_copy(posu.at[pl.ds(base, BPW)], uidx_v.at[pl.ds(0, BPW)])
    pltpu.sync_copy(negu.at[pl.ds(base, BPW)], uidx_v.at[pl.ds(BPW, BPW)])

    m_pos = [mv[0, pl.ds(k * L, L)] for k in range(D // L)]
    m_neg = [mv[1, pl.ds(k * L, L)] for k in range(D // L)]
    iota16 = lax.iota(jnp.int32, L)
    masks = [iota16 == j for j in range(L)]

    def run_table(toff, m, zref):
        _fire_chunk(utab, uidx_v, toff, bufs[0], sems[0])
        _fire_chunk(utab, uidx_v, toff + CH, bufs[1], sems[1])

        def step(i, carry):
            for half in (0, 1):
                c = 2 * i + half
                pltpu.make_async_copy(utab.at[pl.ds(0, CH * 8), :],
                                      bufs[half], sems[half]).wait()
                _dot_chunk(bufs[half], uidx_v, toff + c * CH, m, masks,
                           zref, c * CH)

                @pl.when(c + 2 < NCHUNK)
                def _():
                    _fire_chunk(utab, uidx_v, toff + (c + 2) * CH,
                                bufs[half], sems[half])
            return carry
        lax.fori_loop(0, NCHUNK // 2, step, 0)

    run_table(0, m_pos, zp)
    pltpu.sync_copy(zp, zpos_out.at[pl.ds(base, BPW)])
    run_table(BPW, m_neg, zn)
    pltpu.sync_copy(zn, zneg_out.at[pl.ds(base, BPW)])


_stage_a = functools.partial(
    pl.kernel,
    out_type=(jax.ShapeDtypeStruct((B,), jnp.float32),
              jax.ShapeDtypeStruct((B,), jnp.float32)),
    mesh=plsc.VectorSubcoreMesh(core_axis_name="c", subcore_axis_name="s",
                                num_cores=NC, num_subcores=NS),
    compiler_params=pltpu.CompilerParams(needs_layout_passes=False,
                                         use_tc_tiling_on_sc=True),
    scratch_types=[
        pltpu.VMEM((2 * BPW,), jnp.int32),
        pltpu.VMEM((2, D), jnp.float32),
        pltpu.VMEM((CH * 8, D), jnp.float32),
        pltpu.VMEM((CH * 8, D), jnp.float32),
        pltpu.VMEM((BPW,), jnp.float32),
        pltpu.VMEM((BPW,), jnp.float32),
        pltpu.SemaphoreType.DMA,
        pltpu.SemaphoreType.DMA,
    ],
)(_stage_a_body)


def _loss_body(pz_ref, nz_ref, o_ref):
    def logsig(x):
        return jnp.minimum(x, 0.0) - jnp.log1p(jnp.exp(-jnp.abs(x)))
    total = -(jnp.sum(logsig(pz_ref[...])) + jnp.sum(logsig(-nz_ref[...])))
    o_ref[...] = jnp.reshape(total, (1, 1))


_loss = pl.pallas_call(
    _loss_body,
    out_shape=jax.ShapeDtypeStruct((1, 1), jnp.float32),
)


def kernel(pos_v, pos_u, neg_v, neg_u, v_table, u_table):
    vidx = jnp.concatenate([pos_v[-1], neg_v[-1]])
    means = _mean(vidx, v_table.T)
    zp, zn = _stage_a(means, pos_u, neg_u, u_table)
    out = _loss(zp.reshape(B // 128, 128), zn.reshape(B // 128, 128))
    return out[0, 0]
